# baseline (speedup 1.0000x reference)
import functools
import jax
import jax.numpy as jnp
from jax import lax
from jax.experimental import pallas as pl
from jax.experimental.pallas import tpu as pltpu
from jax.experimental.pallas import tpu_sc as plsc

HD = 128
NHEADS = 8
NS = 16
W1 = 32
W2 = 48

RHALF = 5000
RPAD = 5120
RZ = RPAD // NS
MHALF = 4096
MPAD = 4224
MZ = MPAD // NS

_mesh = plsc.VectorSubcoreMesh(core_axis_name="c", subcore_axis_name="s")


def _pad_to(x, n):
    pad = n - x.shape[0]
    return jnp.concatenate([x, jnp.zeros((pad,) + x.shape[1:], x.dtype)])


def _zero_accs(zeros, accs, s):
    for acc, rz in accs:
        pltpu.sync_copy(zeros.at[pl.ds(s * rz, rz)], acc.at[pl.ds(s * rz, rz)])
    plsc.subcore_barrier()


def _pipe3(nwin, fire_ins, wait_ins, fire_gath, wait_gath, do_dloc,
           do_compute, fire_scat, wait_scat):
    fire_ins(0, 0)
    fire_ins(1, 1)
    wait_ins(0, 0)
    fire_gath(0, 0)

    def outer(g, _):
        for k in range(3):
            i = g * 3 + k
            kp1, kp2 = (k + 1) % 3, (k + 2) % 3

            @pl.when(i + 2 < nwin)
            def _(i=i, kp2=kp2):
                fire_ins(i + 2, kp2)

            @pl.when(i + 1 < nwin)
            def _(i=i, kp1=kp1):
                wait_ins(i + 1, kp1)

            @pl.when(i >= 2)
            def _(i=i, kp1=kp1):
                wait_scat(i - 2, kp1)

            @pl.when(i + 1 < nwin)
            def _(i=i, kp1=kp1):
                fire_gath(i + 1, kp1)

            do_dloc(i, k)
            wait_gath(i, k)
            do_compute(i, k)
            fire_scat(i, k)
        return 0

    lax.fori_loop(0, nwin // 3, outer, 0)
    wait_scat(nwin - 2, (nwin - 2) % 3)
    wait_scat(nwin - 1, (nwin - 1) % 3)


def _mk_dloc(dstv, dloc, rbase, rhalf, w_):
    def do_dloc(i, k):
        garb = rhalf + lax.iota(jnp.int32, 16)
        for kk in range(w_ // 16):
            sl = pl.ds(kk * 16, 16)
            rel = dstv.at[k][sl] - rbase
            inr = (rel >= 0) & (rel < rhalf)
            dloc.at[k][sl] = jnp.where(inr, rel, garb)
    return do_dloc


def _mk_lin_io(refs, bufs, ebase, w_, sem):
    def fire(i, k):
        base = ebase + i * w_
        for r, b in zip(refs, bufs):
            pltpu.async_copy(r.at[pl.ds(base, w_)], b.at[k], sem)

    def wait(i, k):
        base = ebase + i * w_
        for r, b in zip(refs, bufs):
            pltpu.make_async_copy(r.at[pl.ds(base, w_)], b.at[k], sem).wait()

    return fire, wait


def _mk_gath(pairs, sem):
    def fire(i, k):
        for t, ib, db in pairs:
            pltpu.async_copy(t.at[ib.at[k]], db.at[k], sem)

    def wait(i, k):
        for t, ib, db in pairs:
            pltpu.make_async_copy(t.at[ib.at[k]], db.at[k], sem).wait()

    return fire, wait


def _mk_scat(pairs, dloc, sem):
    def fire(i, k):
        for vb, acc in pairs:
            pltpu.async_copy(vb.at[k], acc.at[dloc.at[k]], sem, add=True)

    def wait(i, k):
        for vb, acc in pairs:
            pltpu.make_async_copy(vb.at[k], acc.at[dloc.at[k]], sem).wait()

    return fire, wait


def _p1_body(axn, en, src, dst, w, va, zeros, outA, outB,
             accA, accB, srcv, dstv, dloc, wv, vav, enr, axr, bv,
             sem_in, sem_g, sem_sc):
    c = lax.axis_index("c")
    s = lax.axis_index("s")
    _zero_accs(zeros, [(accA, RZ), (accB, RZ)], s)
    rbase = c * RHALF
    ebase = s * (src.shape[0] // NS)
    fire_ins, wait_ins = _mk_lin_io(
        (src, dst, w, va, en), (srcv, dstv, wv, vav, enr), ebase, W1, sem_in)
    fire_g, wait_g = _mk_gath([(axn, srcv, axr)], sem_g)
    do_dloc = _mk_dloc(dstv, dloc, rbase, RHALF, W1)
    fire_sc, wait_sc = _mk_scat([(axr, accA), (bv, accB)], dloc, sem_sc)

    def compute(i, k):
        axv, env, bvv = axr.at[k], enr.at[k], bv.at[k]

        def grp(g, _):
            kb = g * 16
            w16 = wv.at[k][pl.ds(kb, 16)]
            sb16 = w16 * w16 * vav.at[k][pl.ds(kb, 16)]
            for jj in range(16):
                j = kb + jj
                saj = jnp.full((16,), w16[jj], jnp.float32)
                sbj = jnp.full((16,), sb16[jj], jnp.float32)
                for kk in range(HD // 16):
                    sl = pl.ds(kk * 16, 16)
                    t = axv[j, sl] * env[j, sl]
                    axv[j, sl] = saj * t
                    bvv[j, sl] = sbj * t
            return 0

        lax.fori_loop(0, W1 // 16, grp, 0)

    _pipe3(src.shape[0] // NS // W1, fire_ins, wait_ins, fire_g, wait_g,
           do_dloc, compute, fire_sc, wait_sc)
    plsc.subcore_barrier()
    sl = pl.ds(s * RZ, RZ)
    pltpu.sync_copy(accA.at[sl], outA.at[c, sl])
    pltpu.sync_copy(accB.at[sl], outB.at[c, sl])


def _p2_body(axn, en, src, dst, w, va, vb, zeros, outA, outB,
             accA, accB, srcv, dstv, dloc, wv, vav, vbv, enr, axr, bv,
             sem_in, sem_g, sem_sc):
    c = lax.axis_index("c")
    s = lax.axis_index("s")
    _zero_accs(zeros, [(accA, RZ), (accB, RZ)], s)
    rbase = c * RHALF
    ebase = s * (src.shape[0] // NS)
    fire_ins, wait_ins = _mk_lin_io(
        (src, dst, w, va, vb, en), (srcv, dstv, wv, vav, vbv, enr),
        ebase, W1, sem_in)
    fire_g, wait_g = _mk_gath([(axn, srcv, axr)], sem_g)
    do_dloc = _mk_dloc(dstv, dloc, rbase, RHALF, W1)
    fire_sc, wait_sc = _mk_scat([(axr, accA), (bv, accB)], dloc, sem_sc)

    def compute(i, k):
        axv, env, bvv = axr.at[k], enr.at[k], bv.at[k]

        def grp(g, _):
            kb = g * 16
            w16 = wv.at[k][pl.ds(kb, 16)]
            ww16 = w16 * w16
            sa16 = ww16 * vav.at[k][pl.ds(kb, 16)]
            sb16 = ww16 * vbv.at[k][pl.ds(kb, 16)]
            for jj in range(16):
                j = kb + jj
                saj = jnp.full((16,), sa16[jj], jnp.float32)
                sbj = jnp.full((16,), sb16[jj], jnp.float32)
                for kk in range(HD // 16):
                    sl = pl.ds(kk * 16, 16)
                    t = axv[j, sl] * env[j, sl]
                    axv[j, sl] = saj * t
                    bvv[j, sl] = sbj * t
            return 0

        lax.fori_loop(0, W1 // 16, grp, 0)

    _pipe3(src.shape[0] // NS // W1, fire_ins, wait_ins, fire_g, wait_g,
           do_dloc, compute, fire_sc, wait_sc)
    plsc.subcore_barrier()
    sl = pl.ds(s * RZ, RZ)
    pltpu.sync_copy(accA.at[sl], outA.at[c, sl])
    pltpu.sync_copy(accB.at[sl], outB.at[c, sl])


def _p3_body(av0t, av1t, en, src, dst, w, zeros, outA, outB,
             accA, accB, srcv, dstv, dloc, wv, enr, g0, g1,
             sem_in, sem_g, sem_sc):
    c = lax.axis_index("c")
    s = lax.axis_index("s")
    _zero_accs(zeros, [(accA, RZ), (accB, RZ)], s)
    rbase = c * RHALF
    ebase = s * (src.shape[0] // NS)
    fire_ins, wait_ins = _mk_lin_io(
        (src, dst, w, en), (srcv, dstv, wv, enr), ebase, W1, sem_in)
    fire_g, wait_g = _mk_gath([(av0t, srcv, g0), (av1t, srcv, g1)], sem_g)
    do_dloc = _mk_dloc(dstv, dloc, rbase, RHALF, W1)
    fire_sc, wait_sc = _mk_scat([(g0, accA), (g1, accB)], dloc, sem_sc)

    def compute(i, k):
        g0v, g1v, env = g0.at[k], g1.at[k], enr.at[k]

        def grp(g, _):
            kb = g * 16
            w16 = wv.at[k][pl.ds(kb, 16)]
            for jj in range(16):
                j = kb + jj
                wj = jnp.full((16,), w16[jj], jnp.float32)
                for kk in range(HD // 16):
                    sl = pl.ds(kk * 16, 16)
                    e = wj * env[j, sl]
                    g0v[j, sl] = g0v[j, sl] * e
                    g1v[j, sl] = g1v[j, sl] * e
            return 0

        lax.fori_loop(0, W1 // 16, grp, 0)

    _pipe3(src.shape[0] // NS // W1, fire_ins, wait_ins, fire_g, wait_g,
           do_dloc, compute, fire_sc, wait_sc)
    plsc.subcore_barrier()
    sl = pl.ds(s * RZ, RZ)
    pltpu.sync_copy(accA.at[sl], outA.at[c, sl])
    pltpu.sync_copy(accB.at[sl], outB.at[c, sl])


def _one_chan_phase(table, lin, sr, ds_, wr, acc, rbase, rhalf,
                    srcv, dstv, dloc, wv, enr, g0, sems, s):
    sem_in, sem_g, sem_sc = sems
    ebase = s * (sr.shape[0] // NS)
    fire_ins, wait_ins = _mk_lin_io(
        (sr, ds_, wr, lin), (srcv, dstv, wv, enr), ebase, W2, sem_in)
    fire_g, wait_g = _mk_gath([(table, srcv, g0)], sem_g)
    do_dloc = _mk_dloc(dstv, dloc, rbase, rhalf, W2)
    fire_sc, wait_sc = _mk_scat([(g0, acc)], dloc, sem_sc)

    def compute(i, k):
        g0v, env = g0.at[k], enr.at[k]

        def grp(g, _):
            kb = g * 16
            w16 = wv.at[k][pl.ds(kb, 16)]
            for jj in range(16):
                j = kb + jj
                wj = jnp.full((16,), w16[jj], jnp.float32)
                for kk in range(HD // 16):
                    sl = pl.ds(kk * 16, 16)
                    g0v[j, sl] = wj * g0v[j, sl] * env[j, sl]
            return 0

        lax.fori_loop(0, W2 // 16, grp, 0)

    _pipe3(sr.shape[0] // NS // W2, fire_ins, wait_ins, fire_g, wait_g,
           do_dloc, compute, fire_sc, wait_sc)


def _p4_body(av2t, en, src, dst, w, mxn, attrm, srcm, dstm, wm, zeros,
             outA, outB, accA, accB, srcv, dstv, dloc, wv, enr, g0,
             sem_in, sem_g, sem_sc):
    c = lax.axis_index("c")
    s = lax.axis_index("s")
    _zero_accs(zeros, [(accA, RZ), (accB, RZ)], s)
    sems = (sem_in, sem_g, sem_sc)
    _one_chan_phase(av2t, en, src, dst, w, accA, c * RHALF, RHALF,
                    srcv, dstv, dloc, wv, enr, g0, sems, s)
    _one_chan_phase(mxn, attrm, srcm, dstm, wm, accB, c * RHALF, RHALF,
                    srcv, dstv, dloc, wv, enr, g0, sems, s)
    plsc.subcore_barrier()
    sl = pl.ds(s * RZ, RZ)
    pltpu.sync_copy(accA.at[sl], outA.at[c, sl])
    pltpu.sync_copy(accB.at[sl], outB.at[c, sl])


def _p5_body(axn, attra, srca, dsta, wa, srcp, dstp, zeros,
             outA, pout, accA, srcv, dstv, dloc, wv, enr, g0, g1,
             sem_in, sem_g, sem_sc):
    c = lax.axis_index("c")
    s = lax.axis_index("s")
    _zero_accs(zeros, [(accA, MZ)], s)
    sems = (sem_in, sem_g, sem_sc)
    _one_chan_phase(axn, attra, srca, dsta, wa, accA, c * MHALF, MHALF,
                    srcv, dstv, dloc, wv, enr, g0, sems, s)

    wid = s * 2 + c
    pbase = wid * (srcp.shape[0] // (2 * NS))
    fire_ins, wait_ins = _mk_lin_io((srcp, dstp), (srcv, dstv),
                                    pbase, W2, sem_in)
    fire_g, wait_g = _mk_gath([(axn, srcv, g0), (axn, dstv, g1)], sem_g)

    def no_dloc(i, k):
        pass

    def compute(i, k):
        g0v, g1v = g0.at[k], g1.at[k]

        def grp(g, _):
            kb = g * 16
            for jj in range(16):
                j = kb + jj
                for kk in range(HD // 16):
                    sl = pl.ds(kk * 16, 16)
                    g0v[j, sl] = g0v[j, sl] * g1v[j, sl]
            return 0

        lax.fori_loop(0, W2 // 16, grp, 0)

    def fire_sc(i, k):
        base = pbase + i * W2
        pltpu.async_copy(g0.at[k], pout.at[pl.ds(base, W2)], sem_sc)

    def wait_sc(i, k):
        base = pbase + i * W2
        pltpu.make_async_copy(g0.at[k], pout.at[pl.ds(base, W2)], sem_sc).wait()

    _pipe3(srcp.shape[0] // (2 * NS) // W2, fire_ins, wait_ins, fire_g,
           wait_g, no_dloc, compute, fire_sc, wait_sc)
    plsc.subcore_barrier()
    sl = pl.ds(s * MZ, MZ)
    pltpu.sync_copy(accA.at[sl], outA.at[c, sl])


def _sc_p1(axn, en, src, dst, w, va, zeros):
    return functools.partial(
        pl.kernel,
        out_type=[jax.ShapeDtypeStruct((2, RPAD, HD), jnp.float32)] * 2,
        mesh=_mesh,
        scratch_types=[pltpu.VMEM_SHARED((RPAD, HD), jnp.float32)] * 2
        + [pltpu.VMEM((3, W1), jnp.int32)] * 3
        + [pltpu.VMEM((3, W1), jnp.float32)] * 2
        + [pltpu.VMEM((3, W1, HD), jnp.float32)] * 3
        + [pltpu.SemaphoreType.DMA] * 3,
        name="sc_p1",
    )(_p1_body)(axn, en, src, dst, w, va, zeros)


def _sc_p2(axn, en, src, dst, w, va, vb, zeros):
    return functools.partial(
        pl.kernel,
        out_type=[jax.ShapeDtypeStruct((2, RPAD, HD), jnp.float32)] * 2,
        mesh=_mesh,
        scratch_types=[pltpu.VMEM_SHARED((RPAD, HD), jnp.float32)] * 2
        + [pltpu.VMEM((3, W1), jnp.int32)] * 3
        + [pltpu.VMEM((3, W1), jnp.float32)] * 3
        + [pltpu.VMEM((3, W1, HD), jnp.float32)] * 3
        + [pltpu.SemaphoreType.DMA] * 3,
        name="sc_p2",
    )(_p2_body)(axn, en, src, dst, w, va, vb, zeros)


def _sc_p3(av0t, av1t, en, src, dst, w, zeros):
    return functools.partial(
        pl.kernel,
        out_type=[jax.ShapeDtypeStruct((2, RPAD, HD), jnp.float32)] * 2,
        mesh=_mesh,
        scratch_types=[pltpu.VMEM_SHARED((RPAD, HD), jnp.float32)] * 2
        + [pltpu.VMEM((3, W1), jnp.int32)] * 3
        + [pltpu.VMEM((3, W1), jnp.float32)] * 1
        + [pltpu.VMEM((3, W1, HD), jnp.float32)] * 3
        + [pltpu.SemaphoreType.DMA] * 3,
        name="sc_p3",
    )(_p3_body)(av0t, av1t, en, src, dst, w, zeros)


def _sc_p4(av2t, en, src, dst, w, mxn, attrm, srcm, dstm, wm, zeros):
    return functools.partial(
        pl.kernel,
        out_type=[jax.ShapeDtypeStruct((2, RPAD, HD), jnp.float32)] * 2,
        mesh=_mesh,
        scratch_types=[pltpu.VMEM_SHARED((RPAD, HD), jnp.float32)] * 2
        + [pltpu.VMEM((3, W2), jnp.int32)] * 3
        + [pltpu.VMEM((3, W2), jnp.float32)] * 1
        + [pltpu.VMEM((3, W2, HD), jnp.float32)] * 2
        + [pltpu.SemaphoreType.DMA] * 3,
        name="sc_p4",
    )(_p4_body)(av2t, en, src, dst, w, mxn, attrm, srcm, dstm, wm, zeros)


def _sc_p5(axn, attra, srca, dsta, wa, srcp, dstp, zeros):
    ep = srcp.shape[0]
    return functools.partial(
        pl.kernel,
        out_type=[jax.ShapeDtypeStruct((2, MPAD, HD), jnp.float32),
                  jax.ShapeDtypeStruct((ep, HD), jnp.float32)],
        mesh=_mesh,
        scratch_types=[pltpu.VMEM_SHARED((MPAD, HD), jnp.float32)]
        + [pltpu.VMEM((3, W2), jnp.int32)] * 3
        + [pltpu.VMEM((3, W2), jnp.float32)] * 1
        + [pltpu.VMEM((3, W2, HD), jnp.float32)] * 3
        + [pltpu.SemaphoreType.DMA] * 3,
        name="sc_p5",
    )(_p5_body)(axn, attra, srca, dsta, wa, srcp, dstp, zeros)


def _ln_body(x_ref, g_ref, b_ref, o_ref):
    x = x_ref[...]
    mu = jnp.mean(x, axis=-1, keepdims=True)
    xc = x - mu
    var = jnp.mean(xc * xc, axis=-1, keepdims=True)
    o_ref[...] = xc * lax.rsqrt(var + 1e-5) * g_ref[...] + b_ref[...]


def _tc_ln(x, g, b, block_rows):
    n = x.shape[0]
    return pl.pallas_call(
        _ln_body,
        grid=(n // block_rows,),
        in_specs=[
            pl.BlockSpec((block_rows, HD), lambda i: (i, 0)),
            pl.BlockSpec((1, HD), lambda i: (0, 0)),
            pl.BlockSpec((1, HD), lambda i: (0, 0)),
        ],
        out_specs=pl.BlockSpec((block_rows, HD), lambda i: (i, 0)),
        out_shape=jax.ShapeDtypeStruct((n, HD), jnp.float32),
    )(x, g.reshape(1, HD), b.reshape(1, HD))


def _axout_body(su_ref, sm_ref, w1_ref, wm_ref, ax_ref, o_ref):
    o_ref[...] = (
        jnp.dot(su_ref[0], w1_ref[...], preferred_element_type=jnp.float32)
        + jnp.dot(sm_ref[0], wm_ref[...], preferred_element_type=jnp.float32)
        + ax_ref[...]
    )


def _tc_axout(seg_u, seg_m2a, W_s1, W_m2a, a_x, block_rows):
    n = a_x.shape[0]
    npb = RHALF // block_rows
    hspec = pl.BlockSpec((1, block_rows, HD), lambda i: (i // npb, i % npb, 0))
    bs = pl.BlockSpec((block_rows, HD), lambda i: (i, 0))
    wspec = pl.BlockSpec((HD, HD), lambda i: (0, 0))
    return pl.pallas_call(
        _axout_body,
        grid=(n // block_rows,),
        in_specs=[hspec, hspec, wspec, wspec, bs],
        out_specs=bs,
        out_shape=jax.ShapeDtypeStruct((n, HD), jnp.float32),
    )(seg_u, seg_m2a, W_s1, W_m2a, a_x)


def _avec_body(av0_ref, av1_ref, av2_ref, b0_ref, b1_ref, b2_ref,
               w2_ref, vec_ref, o_ref):
    w2 = w2_ref[...]
    for c, (avr, br) in enumerate(((av0_ref, b0_ref), (av1_ref, b1_ref),
                                   (av2_ref, b2_ref))):
        o_ref[:, c, :] = (
            avr[0]
            + jnp.dot(br[0], w2, preferred_element_type=jnp.float32)
            + vec_ref[:, c, :]
        )


def _tc_avecout(av0, av1, av2, b0, b1, b2, W_s2, a_vec, block_rows):
    n = a_vec.shape[0]
    npb = RHALF // block_rows
    hspec = pl.BlockSpec((1, block_rows, HD), lambda i: (i // npb, i % npb, 0))
    return pl.pallas_call(
        _avec_body,
        grid=(n // block_rows,),
        in_specs=[hspec] * 6 + [
            pl.BlockSpec((HD, HD), lambda i: (0, 0)),
            pl.BlockSpec((block_rows, 3, HD), lambda i: (i, 0, 0)),
        ],
        out_specs=pl.BlockSpec((block_rows, 3, HD), lambda i: (i, 0, 0)),
        out_shape=jax.ShapeDtypeStruct((n, 3, HD), jnp.float32),
    )(av0, av1, av2, b0, b1, b2, W_s2, a_vec)


def _se_body(p_ref, we_ref, attr_ref, o_ref):
    o_ref[...] = (
        jnp.dot(p_ref[...], we_ref[...], preferred_element_type=jnp.float32)
        + attr_ref[...]
    )


def _tc_seout(p, W_e, attr, block_rows):
    e = attr.shape[0]
    bs = pl.BlockSpec((block_rows, HD), lambda i: (i, 0))
    return pl.pallas_call(
        _se_body,
        grid=(e // block_rows,),
        in_specs=[bs, pl.BlockSpec((HD, HD), lambda i: (0, 0)), bs],
        out_specs=bs,
        out_shape=jax.ShapeDtypeStruct((e, HD), jnp.float32),
    )(p, W_e, attr)


def _mha_body(x_ref, wq_ref, wk_ref, wv_ref, wo_ref, segm_ref, wam_ref,
              mx_ref, o_ref):
    x = x_ref[...]
    q = jnp.dot(x, wq_ref[...], preferred_element_type=jnp.float32)
    k = jnp.dot(x, wk_ref[...], preferred_element_type=jnp.float32)
    v = jnp.dot(x, wv_ref[...], preferred_element_type=jnp.float32)
    hd = HD // NHEADS
    outs = []
    for h in range(NHEADS):
        qh = q[:, h * hd:(h + 1) * hd]
        kh = k[:, h * hd:(h + 1) * hd]
        vh = v[:, h * hd:(h + 1) * hd]
        s = jax.lax.dot_general(qh, kh, (((1,), (1,)), ((), ())),
                                preferred_element_type=jnp.float32)
        s = s * (1.0 / jnp.sqrt(jnp.float32(hd)))
        m = jnp.max(s, axis=-1, keepdims=True)
        e = jnp.exp(s - m)
        att = e / jnp.sum(e, axis=-1, keepdims=True)
        outs.append(jnp.dot(att, vh, preferred_element_type=jnp.float32))
    o = jnp.concatenate(outs, axis=-1)
    o_ref[...] = (
        jnp.dot(o, wo_ref[...], preferred_element_type=jnp.float32)
        + jnp.dot(segm_ref[0], wam_ref[...], preferred_element_type=jnp.float32)
        + mx_ref[...]
    )


def _tc_mout(m_xn, Wq, Wk, Wv, Wo, seg_a2m, W_a2m, m_x, ng):
    m = m_x.shape[0]
    npb = MHALF // ng
    bs = pl.BlockSpec((ng, HD), lambda i: (i, 0))
    hspec = pl.BlockSpec((1, ng, HD), lambda i: (i // npb, i % npb, 0))
    wspec = pl.BlockSpec((HD, HD), lambda i: (0, 0))
    return pl.pallas_call(
        _mha_body,
        grid=(m // ng,),
        in_specs=[bs, wspec, wspec, wspec, wspec, hspec, wspec, bs],
        out_specs=bs,
        out_shape=jax.ShapeDtypeStruct((m, HD), jnp.float32),
    )(m_xn, Wq, Wk, Wv, Wo, seg_a2m, W_a2m, m_x)


def kernel(a_x, a_vec, m_x, a2a_edge_index, a2m_edge_index, m2a_edge_index,
           a2a_edge_weights, a2m_edge_weights, m2a_edge_weights,
           a2a_edge_attr, a2m_edge_attr, m2a_edge_attr, a2a_edge_vecs,
           W_s1, W_s2, W_e, W_a2m, W_m2a, Wq, Wk, Wv, Wo,
           ln_s_g, ln_s_b, ln_f_g, ln_f_b, ln_l_g, ln_l_b):
    E = a2a_edge_attr.shape[0]
    EAM = a2m_edge_attr.shape[0]
    qa = NS * 3 * W1 * 3
    EP = ((E + qa - 1) // qa) * qa
    qm = NS * W2 * 3
    EAMP = ((EAM + qm - 1) // qm) * qm

    a_xn = _tc_ln(a_x, ln_s_g, ln_s_b, 1000)
    m_xn = _tc_ln(m_x, ln_l_g, ln_l_b, 1024)
    en = _tc_ln(_pad_to(a2a_edge_attr, EP), ln_f_g / HD, ln_f_b / HD, 960)

    srcA = _pad_to(a2a_edge_index[0], EP)
    dstA = _pad_to(a2a_edge_index[1], EP)
    wA = _pad_to(a2a_edge_weights, EP)
    v0 = _pad_to(a2a_edge_vecs[:, 0], EP)
    v1 = _pad_to(a2a_edge_vecs[:, 1], EP)
    v2 = _pad_to(a2a_edge_vecs[:, 2], EP)
    av0t = a_vec[:, 0, :]
    av1t = a_vec[:, 1, :]
    av2t = a_vec[:, 2, :]
    srcM = _pad_to(m2a_edge_index[0], EAMP)
    dstM = _pad_to(m2a_edge_index[1], EAMP)
    wM = _pad_to(m2a_edge_weights, EAMP)
    attrM = _pad_to(m2a_edge_attr, EAMP)
    srcAm = _pad_to(a2m_edge_index[0], EAMP)
    dstAm = _pad_to(a2m_edge_index[1], EAMP)
    wAm = _pad_to(a2m_edge_weights, EAMP)
    attrAm = _pad_to(a2m_edge_attr, EAMP)
    zeros = jnp.zeros((RPAD, HD), jnp.float32)

    segA2M, p = _sc_p5(a_xn, attrAm, srcAm, dstAm, wAm,
                       srcA, dstA, zeros)
    out_e = _tc_seout(p, W_e, a2a_edge_attr, 1000)
    out_mx = _tc_mout(m_xn, Wq, Wk, Wv, Wo, segA2M, W_a2m, m_x, 512)
    segAV2, segM2A = _sc_p4(av2t, en, srcA, dstA, wA,
                            m_xn, attrM, srcM, dstM, wM, zeros)
    segU, segB0 = _sc_p1(a_xn, en, srcA, dstA, wA, v0, zeros)
    out_ax = _tc_axout(segU, segM2A, W_s1, W_m2a, a_x, 1000)
    segB1, segB2 = _sc_p2(a_xn, en, srcA, dstA, wA, v1, v2, zeros)
    segAV0, segAV1 = _sc_p3(av0t, av1t, en, srcA, dstA, wA, zeros)
    out_avec = _tc_avecout(segAV0, segAV1, segAV2, segB0, segB1, segB2,
                           W_s2, a_vec, 1000)
    return out_ax, out_mx, out_avec, out_e


# speedup vs baseline: 1.0341x; 1.0341x over previous
import functools
import jax
import jax.numpy as jnp
from jax import lax
from jax.experimental import pallas as pl
from jax.experimental.pallas import tpu as pltpu
from jax.experimental.pallas import tpu_sc as plsc

HD = 128
NHEADS = 8
NS = 16
W1 = 32
W2 = 48

RHALF = 5000
RPAD = 5120
RZ = RPAD // NS
MHALF = 4096
MPAD = 4224
MZ = MPAD // NS

_mesh = plsc.VectorSubcoreMesh(core_axis_name="c", subcore_axis_name="s")


def _pad_to(x, n):
    pad = n - x.shape[0]
    return jnp.concatenate([x, jnp.zeros((pad,) + x.shape[1:], x.dtype)])


def _zero_accs(zeros, accs, s):
    for acc, rz in accs:
        pltpu.sync_copy(zeros.at[pl.ds(s * rz, rz)], acc.at[pl.ds(s * rz, rz)])
    plsc.subcore_barrier()


def _pipe3(nwin, fire_ins, wait_ins, fire_gath, wait_gath, do_dloc,
           do_compute, fire_scat, wait_scat):
    fire_ins(0, 0)
    fire_ins(1, 1)
    wait_ins(0, 0)
    fire_gath(0, 0)

    def outer(g, _):
        for k in range(3):
            i = g * 3 + k
            kp1, kp2 = (k + 1) % 3, (k + 2) % 3

            @pl.when(i + 2 < nwin)
            def _(i=i, kp2=kp2):
                fire_ins(i + 2, kp2)

            @pl.when(i + 1 < nwin)
            def _(i=i, kp1=kp1):
                wait_ins(i + 1, kp1)

            @pl.when(i >= 2)
            def _(i=i, kp1=kp1):
                wait_scat(i - 2, kp1)

            @pl.when(i + 1 < nwin)
            def _(i=i, kp1=kp1):
                fire_gath(i + 1, kp1)

            do_dloc(i, k)
            wait_gath(i, k)
            do_compute(i, k)
            fire_scat(i, k)
        return 0

    lax.fori_loop(0, nwin // 3, outer, 0)
    wait_scat(nwin - 2, (nwin - 2) % 3)
    wait_scat(nwin - 1, (nwin - 1) % 3)


def _mk_dloc(dstv, dloc, rbase, rhalf, w_):
    def do_dloc(i, k):
        garb = rhalf + lax.iota(jnp.int32, 16)
        for kk in range(w_ // 16):
            sl = pl.ds(kk * 16, 16)
            rel = dstv.at[k][sl] - rbase
            inr = (rel >= 0) & (rel < rhalf)
            dloc.at[k][sl] = jnp.where(inr, rel, garb)
    return do_dloc


def _mk_lin_io(refs, bufs, ebase, w_, sem, limits=None):
    if limits is None:
        limits = (None,) * len(refs)

    def _base(i, lim):
        base = ebase + i * w_
        return base if lim is None else jnp.minimum(base, lim)

    def fire(i, k):
        for r, b, lim in zip(refs, bufs, limits):
            pltpu.async_copy(r.at[pl.ds(_base(i, lim), w_)], b.at[k], sem)

    def wait(i, k):
        for r, b, lim in zip(refs, bufs, limits):
            pltpu.make_async_copy(
                r.at[pl.ds(_base(i, lim), w_)], b.at[k], sem).wait()

    return fire, wait


def _mk_gath(pairs, sem):
    def fire(i, k):
        for t, ib, db in pairs:
            pltpu.async_copy(t.at[ib.at[k]], db.at[k], sem)

    def wait(i, k):
        for t, ib, db in pairs:
            pltpu.make_async_copy(t.at[ib.at[k]], db.at[k], sem).wait()

    return fire, wait


def _mk_scat(pairs, dloc, sem):
    def fire(i, k):
        for vb, acc in pairs:
            pltpu.async_copy(vb.at[k], acc.at[dloc.at[k]], sem, add=True)

    def wait(i, k):
        for vb, acc in pairs:
            pltpu.make_async_copy(vb.at[k], acc.at[dloc.at[k]], sem).wait()

    return fire, wait


def _p1_body(axn, en, src, dst, w, va, zeros, outA, outB,
             accA, accB, srcv, dstv, dloc, wv, vav, enr, axr, bv,
             sem_in, sem_g, sem_sc):
    c = lax.axis_index("c")
    s = lax.axis_index("s")
    _zero_accs(zeros, [(accA, RZ), (accB, RZ)], s)
    rbase = c * RHALF
    ebase = s * (src.shape[0] // NS)
    fire_ins, wait_ins = _mk_lin_io(
        (src, dst, w, va, en), (srcv, dstv, wv, vav, enr), ebase, W1, sem_in,
        limits=(None, None, None, None, en.shape[0] - W1))
    fire_g, wait_g = _mk_gath([(axn, srcv, axr)], sem_g)
    do_dloc = _mk_dloc(dstv, dloc, rbase, RHALF, W1)
    fire_sc, wait_sc = _mk_scat([(axr, accA), (bv, accB)], dloc, sem_sc)

    def compute(i, k):
        axv, env, bvv = axr.at[k], enr.at[k], bv.at[k]

        def grp(g, _):
            kb = g * 16
            w16 = wv.at[k][pl.ds(kb, 16)]
            sb16 = w16 * w16 * vav.at[k][pl.ds(kb, 16)]
            for jj in range(16):
                j = kb + jj
                saj = jnp.full((16,), w16[jj], jnp.float32)
                sbj = jnp.full((16,), sb16[jj], jnp.float32)
                for kk in range(HD // 16):
                    sl = pl.ds(kk * 16, 16)
                    t = axv[j, sl] * env[j, sl]
                    axv[j, sl] = saj * t
                    bvv[j, sl] = sbj * t
            return 0

        lax.fori_loop(0, W1 // 16, grp, 0)

    _pipe3(src.shape[0] // NS // W1, fire_ins, wait_ins, fire_g, wait_g,
           do_dloc, compute, fire_sc, wait_sc)
    plsc.subcore_barrier()
    sl = pl.ds(s * RZ, RZ)
    pltpu.sync_copy(accA.at[sl], outA.at[c, sl])
    pltpu.sync_copy(accB.at[sl], outB.at[c, sl])


def _p2_body(axn, en, src, dst, w, va, vb, zeros, outA, outB,
             accA, accB, srcv, dstv, dloc, wv, vav, vbv, enr, axr, bv,
             sem_in, sem_g, sem_sc):
    c = lax.axis_index("c")
    s = lax.axis_index("s")
    _zero_accs(zeros, [(accA, RZ), (accB, RZ)], s)
    rbase = c * RHALF
    ebase = s * (src.shape[0] // NS)
    fire_ins, wait_ins = _mk_lin_io(
        (src, dst, w, va, vb, en), (srcv, dstv, wv, vav, vbv, enr),
        ebase, W1, sem_in,
        limits=(None, None, None, None, None, en.shape[0] - W1))
    fire_g, wait_g = _mk_gath([(axn, srcv, axr)], sem_g)
    do_dloc = _mk_dloc(dstv, dloc, rbase, RHALF, W1)
    fire_sc, wait_sc = _mk_scat([(axr, accA), (bv, accB)], dloc, sem_sc)

    def compute(i, k):
        axv, env, bvv = axr.at[k], enr.at[k], bv.at[k]

        def grp(g, _):
            kb = g * 16
            w16 = wv.at[k][pl.ds(kb, 16)]
            ww16 = w16 * w16
            sa16 = ww16 * vav.at[k][pl.ds(kb, 16)]
            sb16 = ww16 * vbv.at[k][pl.ds(kb, 16)]
            for jj in range(16):
                j = kb + jj
                saj = jnp.full((16,), sa16[jj], jnp.float32)
                sbj = jnp.full((16,), sb16[jj], jnp.float32)
                for kk in range(HD // 16):
                    sl = pl.ds(kk * 16, 16)
                    t = axv[j, sl] * env[j, sl]
                    axv[j, sl] = saj * t
                    bvv[j, sl] = sbj * t
            return 0

        lax.fori_loop(0, W1 // 16, grp, 0)

    _pipe3(src.shape[0] // NS // W1, fire_ins, wait_ins, fire_g, wait_g,
           do_dloc, compute, fire_sc, wait_sc)
    plsc.subcore_barrier()
    sl = pl.ds(s * RZ, RZ)
    pltpu.sync_copy(accA.at[sl], outA.at[c, sl])
    pltpu.sync_copy(accB.at[sl], outB.at[c, sl])


def _p3_body(av0t, av1t, en, src, dst, w, zeros, outA, outB,
             accA, accB, srcv, dstv, dloc, wv, enr, g0, g1,
             sem_in, sem_g, sem_sc):
    c = lax.axis_index("c")
    s = lax.axis_index("s")
    _zero_accs(zeros, [(accA, RZ), (accB, RZ)], s)
    rbase = c * RHALF
    ebase = s * (src.shape[0] // NS)
    fire_ins, wait_ins = _mk_lin_io(
        (src, dst, w, en), (srcv, dstv, wv, enr), ebase, W1, sem_in,
        limits=(None, None, None, en.shape[0] - W1))
    fire_g, wait_g = _mk_gath([(av0t, srcv, g0), (av1t, srcv, g1)], sem_g)
    do_dloc = _mk_dloc(dstv, dloc, rbase, RHALF, W1)
    fire_sc, wait_sc = _mk_scat([(g0, accA), (g1, accB)], dloc, sem_sc)

    def compute(i, k):
        g0v, g1v, env = g0.at[k], g1.at[k], enr.at[k]

        def grp(g, _):
            kb = g * 16
            w16 = wv.at[k][pl.ds(kb, 16)]
            for jj in range(16):
                j = kb + jj
                wj = jnp.full((16,), w16[jj], jnp.float32)
                for kk in range(HD // 16):
                    sl = pl.ds(kk * 16, 16)
                    e = wj * env[j, sl]
                    g0v[j, sl] = g0v[j, sl] * e
                    g1v[j, sl] = g1v[j, sl] * e
            return 0

        lax.fori_loop(0, W1 // 16, grp, 0)

    _pipe3(src.shape[0] // NS // W1, fire_ins, wait_ins, fire_g, wait_g,
           do_dloc, compute, fire_sc, wait_sc)
    plsc.subcore_barrier()
    sl = pl.ds(s * RZ, RZ)
    pltpu.sync_copy(accA.at[sl], outA.at[c, sl])
    pltpu.sync_copy(accB.at[sl], outB.at[c, sl])


def _one_chan_phase(table, lin, sr, ds_, wr, acc, rbase, rhalf,
                    srcv, dstv, dloc, wv, enr, g0, sems, s):
    sem_in, sem_g, sem_sc = sems
    ebase = s * (sr.shape[0] // NS)
    fire_ins, wait_ins = _mk_lin_io(
        (sr, ds_, wr, lin), (srcv, dstv, wv, enr), ebase, W2, sem_in,
        limits=(None, None, None, lin.shape[0] - W2))
    fire_g, wait_g = _mk_gath([(table, srcv, g0)], sem_g)
    do_dloc = _mk_dloc(dstv, dloc, rbase, rhalf, W2)
    fire_sc, wait_sc = _mk_scat([(g0, acc)], dloc, sem_sc)

    def compute(i, k):
        g0v, env = g0.at[k], enr.at[k]

        def grp(g, _):
            kb = g * 16
            w16 = wv.at[k][pl.ds(kb, 16)]
            for jj in range(16):
                j = kb + jj
                wj = jnp.full((16,), w16[jj], jnp.float32)
                for kk in range(HD // 16):
                    sl = pl.ds(kk * 16, 16)
                    g0v[j, sl] = wj * g0v[j, sl] * env[j, sl]
            return 0

        lax.fori_loop(0, W2 // 16, grp, 0)

    _pipe3(sr.shape[0] // NS // W2, fire_ins, wait_ins, fire_g, wait_g,
           do_dloc, compute, fire_sc, wait_sc)


def _p4_body(av2t, en, src, dst, w, mxn, attrm, srcm, dstm, wm, zeros,
             outA, outB, accA, accB, srcv, dstv, dloc, wv, enr, g0,
             sem_in, sem_g, sem_sc):
    c = lax.axis_index("c")
    s = lax.axis_index("s")
    _zero_accs(zeros, [(accA, RZ), (accB, RZ)], s)
    sems = (sem_in, sem_g, sem_sc)
    _one_chan_phase(av2t, en, src, dst, w, accA, c * RHALF, RHALF,
                    srcv, dstv, dloc, wv, enr, g0, sems, s)
    _one_chan_phase(mxn, attrm, srcm, dstm, wm, accB, c * RHALF, RHALF,
                    srcv, dstv, dloc, wv, enr, g0, sems, s)
    plsc.subcore_barrier()
    sl = pl.ds(s * RZ, RZ)
    pltpu.sync_copy(accA.at[sl], outA.at[c, sl])
    pltpu.sync_copy(accB.at[sl], outB.at[c, sl])


def _p5_body(axn, attra, srca, dsta, wa, srcp, dstp, zeros,
             outA, pout, accA, srcv, dstv, dloc, wv, enr, g0, g1,
             sem_in, sem_g, sem_sc):
    c = lax.axis_index("c")
    s = lax.axis_index("s")
    _zero_accs(zeros, [(accA, MZ)], s)
    sems = (sem_in, sem_g, sem_sc)
    _one_chan_phase(axn, attra, srca, dsta, wa, accA, c * MHALF, MHALF,
                    srcv, dstv, dloc, wv, enr, g0, sems, s)

    wid = s * 2 + c
    pbase = wid * (srcp.shape[0] // (2 * NS))
    fire_ins, wait_ins = _mk_lin_io((srcp, dstp), (srcv, dstv),
                                    pbase, W2, sem_in)
    fire_g, wait_g = _mk_gath([(axn, srcv, g0), (axn, dstv, g1)], sem_g)

    def no_dloc(i, k):
        pass

    def compute(i, k):
        g0v, g1v = g0.at[k], g1.at[k]

        def grp(g, _):
            kb = g * 16
            for jj in range(16):
                j = kb + jj
                for kk in range(HD // 16):
                    sl = pl.ds(kk * 16, 16)
                    g0v[j, sl] = g0v[j, sl] * g1v[j, sl]
            return 0

        lax.fori_loop(0, W2 // 16, grp, 0)

    def fire_sc(i, k):
        base = pbase + i * W2
        pltpu.async_copy(g0.at[k], pout.at[pl.ds(base, W2)], sem_sc)

    def wait_sc(i, k):
        base = pbase + i * W2
        pltpu.make_async_copy(g0.at[k], pout.at[pl.ds(base, W2)], sem_sc).wait()

    _pipe3(srcp.shape[0] // (2 * NS) // W2, fire_ins, wait_ins, fire_g,
           wait_g, no_dloc, compute, fire_sc, wait_sc)
    plsc.subcore_barrier()
    sl = pl.ds(s * MZ, MZ)
    pltpu.sync_copy(accA.at[sl], outA.at[c, sl])


def _sc_p1(axn, en, src, dst, w, va, zeros):
    return functools.partial(
        pl.kernel,
        out_type=[jax.ShapeDtypeStruct((2, RPAD, HD), jnp.float32)] * 2,
        mesh=_mesh,
        scratch_types=[pltpu.VMEM_SHARED((RPAD, HD), jnp.float32)] * 2
        + [pltpu.VMEM((3, W1), jnp.int32)] * 3
        + [pltpu.VMEM((3, W1), jnp.float32)] * 2
        + [pltpu.VMEM((3, W1, HD), jnp.float32)] * 3
        + [pltpu.SemaphoreType.DMA] * 3,
        name="sc_p1",
    )(_p1_body)(axn, en, src, dst, w, va, zeros)


def _sc_p2(axn, en, src, dst, w, va, vb, zeros):
    return functools.partial(
        pl.kernel,
        out_type=[jax.ShapeDtypeStruct((2, RPAD, HD), jnp.float32)] * 2,
        mesh=_mesh,
        scratch_types=[pltpu.VMEM_SHARED((RPAD, HD), jnp.float32)] * 2
        + [pltpu.VMEM((3, W1), jnp.int32)] * 3
        + [pltpu.VMEM((3, W1), jnp.float32)] * 3
        + [pltpu.VMEM((3, W1, HD), jnp.float32)] * 3
        + [pltpu.SemaphoreType.DMA] * 3,
        name="sc_p2",
    )(_p2_body)(axn, en, src, dst, w, va, vb, zeros)


def _sc_p3(av0t, av1t, en, src, dst, w, zeros):
    return functools.partial(
        pl.kernel,
        out_type=[jax.ShapeDtypeStruct((2, RPAD, HD), jnp.float32)] * 2,
        mesh=_mesh,
        scratch_types=[pltpu.VMEM_SHARED((RPAD, HD), jnp.float32)] * 2
        + [pltpu.VMEM((3, W1), jnp.int32)] * 3
        + [pltpu.VMEM((3, W1), jnp.float32)] * 1
        + [pltpu.VMEM((3, W1, HD), jnp.float32)] * 3
        + [pltpu.SemaphoreType.DMA] * 3,
        name="sc_p3",
    )(_p3_body)(av0t, av1t, en, src, dst, w, zeros)


def _sc_p4(av2t, en, src, dst, w, mxn, attrm, srcm, dstm, wm, zeros):
    return functools.partial(
        pl.kernel,
        out_type=[jax.ShapeDtypeStruct((2, RPAD, HD), jnp.float32)] * 2,
        mesh=_mesh,
        scratch_types=[pltpu.VMEM_SHARED((RPAD, HD), jnp.float32)] * 2
        + [pltpu.VMEM((3, W2), jnp.int32)] * 3
        + [pltpu.VMEM((3, W2), jnp.float32)] * 1
        + [pltpu.VMEM((3, W2, HD), jnp.float32)] * 2
        + [pltpu.SemaphoreType.DMA] * 3,
        name="sc_p4",
    )(_p4_body)(av2t, en, src, dst, w, mxn, attrm, srcm, dstm, wm, zeros)


def _sc_p5(axn, attra, srca, dsta, wa, srcp, dstp, zeros):
    ep = srcp.shape[0]
    return functools.partial(
        pl.kernel,
        out_type=[jax.ShapeDtypeStruct((2, MPAD, HD), jnp.float32),
                  jax.ShapeDtypeStruct((ep, HD), jnp.float32)],
        mesh=_mesh,
        scratch_types=[pltpu.VMEM_SHARED((MPAD, HD), jnp.float32)]
        + [pltpu.VMEM((3, W2), jnp.int32)] * 3
        + [pltpu.VMEM((3, W2), jnp.float32)] * 1
        + [pltpu.VMEM((3, W2, HD), jnp.float32)] * 3
        + [pltpu.SemaphoreType.DMA] * 3,
        name="sc_p5",
    )(_p5_body)(axn, attra, srca, dsta, wa, srcp, dstp, zeros)


def _ln_body(x_ref, g_ref, b_ref, o_ref):
    x = x_ref[...]
    mu = jnp.mean(x, axis=-1, keepdims=True)
    xc = x - mu
    var = jnp.mean(xc * xc, axis=-1, keepdims=True)
    o_ref[...] = xc * lax.rsqrt(var + 1e-5) * g_ref[...] + b_ref[...]


def _tc_ln(x, g, b, block_rows):
    n = x.shape[0]
    return pl.pallas_call(
        _ln_body,
        grid=(n // block_rows,),
        in_specs=[
            pl.BlockSpec((block_rows, HD), lambda i: (i, 0)),
            pl.BlockSpec((1, HD), lambda i: (0, 0)),
            pl.BlockSpec((1, HD), lambda i: (0, 0)),
        ],
        out_specs=pl.BlockSpec((block_rows, HD), lambda i: (i, 0)),
        out_shape=jax.ShapeDtypeStruct((n, HD), jnp.float32),
    )(x, g.reshape(1, HD), b.reshape(1, HD))


def _axout_body(su_ref, sm_ref, w1_ref, wm_ref, ax_ref, o_ref):
    o_ref[...] = (
        jnp.dot(su_ref[0], w1_ref[...], preferred_element_type=jnp.float32)
        + jnp.dot(sm_ref[0], wm_ref[...], preferred_element_type=jnp.float32)
        + ax_ref[...]
    )


def _tc_axout(seg_u, seg_m2a, W_s1, W_m2a, a_x, block_rows):
    n = a_x.shape[0]
    npb = RHALF // block_rows
    hspec = pl.BlockSpec((1, block_rows, HD), lambda i: (i // npb, i % npb, 0))
    bs = pl.BlockSpec((block_rows, HD), lambda i: (i, 0))
    wspec = pl.BlockSpec((HD, HD), lambda i: (0, 0))
    return pl.pallas_call(
        _axout_body,
        grid=(n // block_rows,),
        in_specs=[hspec, hspec, wspec, wspec, bs],
        out_specs=bs,
        out_shape=jax.ShapeDtypeStruct((n, HD), jnp.float32),
    )(seg_u, seg_m2a, W_s1, W_m2a, a_x)


def _avec_body(av0_ref, av1_ref, av2_ref, b0_ref, b1_ref, b2_ref,
               w2_ref, vec_ref, o_ref):
    w2 = w2_ref[...]
    for c, (avr, br) in enumerate(((av0_ref, b0_ref), (av1_ref, b1_ref),
                                   (av2_ref, b2_ref))):
        o_ref[:, c, :] = (
            avr[0]
            + jnp.dot(br[0], w2, preferred_element_type=jnp.float32)
            + vec_ref[:, c, :]
        )


def _tc_avecout(av0, av1, av2, b0, b1, b2, W_s2, a_vec, block_rows):
    n = a_vec.shape[0]
    npb = RHALF // block_rows
    hspec = pl.BlockSpec((1, block_rows, HD), lambda i: (i // npb, i % npb, 0))
    return pl.pallas_call(
        _avec_body,
        grid=(n // block_rows,),
        in_specs=[hspec] * 6 + [
            pl.BlockSpec((HD, HD), lambda i: (0, 0)),
            pl.BlockSpec((block_rows, 3, HD), lambda i: (i, 0, 0)),
        ],
        out_specs=pl.BlockSpec((block_rows, 3, HD), lambda i: (i, 0, 0)),
        out_shape=jax.ShapeDtypeStruct((n, 3, HD), jnp.float32),
    )(av0, av1, av2, b0, b1, b2, W_s2, a_vec)


def _se_body(p_ref, we_ref, attr_ref, o_ref):
    o_ref[...] = (
        jnp.dot(p_ref[...], we_ref[...], preferred_element_type=jnp.float32)
        + attr_ref[...]
    )


def _tc_seout(p, W_e, attr, block_rows):
    e = attr.shape[0]
    bs = pl.BlockSpec((block_rows, HD), lambda i: (i, 0))
    return pl.pallas_call(
        _se_body,
        grid=(e // block_rows,),
        in_specs=[bs, pl.BlockSpec((HD, HD), lambda i: (0, 0)), bs],
        out_specs=bs,
        out_shape=jax.ShapeDtypeStruct((e, HD), jnp.float32),
    )(p, W_e, attr)


def _mha_body(x_ref, wq_ref, wk_ref, wv_ref, wo_ref, segm_ref, wam_ref,
              mx_ref, o_ref):
    x = x_ref[...]
    q = jnp.dot(x, wq_ref[...], preferred_element_type=jnp.float32)
    k = jnp.dot(x, wk_ref[...], preferred_element_type=jnp.float32)
    v = jnp.dot(x, wv_ref[...], preferred_element_type=jnp.float32)
    hd = HD // NHEADS
    outs = []
    for h in range(NHEADS):
        qh = q[:, h * hd:(h + 1) * hd]
        kh = k[:, h * hd:(h + 1) * hd]
        vh = v[:, h * hd:(h + 1) * hd]
        s = jax.lax.dot_general(qh, kh, (((1,), (1,)), ((), ())),
                                preferred_element_type=jnp.float32)
        s = s * (1.0 / jnp.sqrt(jnp.float32(hd)))
        m = jnp.max(s, axis=-1, keepdims=True)
        e = jnp.exp(s - m)
        att = e / jnp.sum(e, axis=-1, keepdims=True)
        outs.append(jnp.dot(att, vh, preferred_element_type=jnp.float32))
    o = jnp.concatenate(outs, axis=-1)
    o_ref[...] = (
        jnp.dot(o, wo_ref[...], preferred_element_type=jnp.float32)
        + jnp.dot(segm_ref[0], wam_ref[...], preferred_element_type=jnp.float32)
        + mx_ref[...]
    )


def _tc_mout(m_xn, Wq, Wk, Wv, Wo, seg_a2m, W_a2m, m_x, ng):
    m = m_x.shape[0]
    npb = MHALF // ng
    bs = pl.BlockSpec((ng, HD), lambda i: (i, 0))
    hspec = pl.BlockSpec((1, ng, HD), lambda i: (i // npb, i % npb, 0))
    wspec = pl.BlockSpec((HD, HD), lambda i: (0, 0))
    return pl.pallas_call(
        _mha_body,
        grid=(m // ng,),
        in_specs=[bs, wspec, wspec, wspec, wspec, hspec, wspec, bs],
        out_specs=bs,
        out_shape=jax.ShapeDtypeStruct((m, HD), jnp.float32),
    )(m_xn, Wq, Wk, Wv, Wo, seg_a2m, W_a2m, m_x)


def kernel(a_x, a_vec, m_x, a2a_edge_index, a2m_edge_index, m2a_edge_index,
           a2a_edge_weights, a2m_edge_weights, m2a_edge_weights,
           a2a_edge_attr, a2m_edge_attr, m2a_edge_attr, a2a_edge_vecs,
           W_s1, W_s2, W_e, W_a2m, W_m2a, Wq, Wk, Wv, Wo,
           ln_s_g, ln_s_b, ln_f_g, ln_f_b, ln_l_g, ln_l_b):
    E = a2a_edge_attr.shape[0]
    EAM = a2m_edge_attr.shape[0]
    qa = NS * 3 * W1 * 3
    EP = ((E + qa - 1) // qa) * qa
    qm = NS * W2 * 3
    EAMP = ((EAM + qm - 1) // qm) * qm

    a_xn = _tc_ln(a_x, ln_s_g, ln_s_b, 1000)
    m_xn = _tc_ln(m_x, ln_l_g, ln_l_b, 1024)
    en = _tc_ln(a2a_edge_attr, ln_f_g / HD, ln_f_b / HD, 1000)

    srcA = _pad_to(a2a_edge_index[0], EP)
    dstA = _pad_to(a2a_edge_index[1], EP)
    wA = _pad_to(a2a_edge_weights, EP)
    v0 = _pad_to(a2a_edge_vecs[:, 0], EP)
    v1 = _pad_to(a2a_edge_vecs[:, 1], EP)
    v2 = _pad_to(a2a_edge_vecs[:, 2], EP)
    av0t = a_vec[:, 0, :]
    av1t = a_vec[:, 1, :]
    av2t = a_vec[:, 2, :]
    srcM = _pad_to(m2a_edge_index[0], EAMP)
    dstM = _pad_to(m2a_edge_index[1], EAMP)
    wM = _pad_to(m2a_edge_weights, EAMP)
    attrM = m2a_edge_attr
    srcAm = _pad_to(a2m_edge_index[0], EAMP)
    dstAm = _pad_to(a2m_edge_index[1], EAMP)
    wAm = _pad_to(a2m_edge_weights, EAMP)
    attrAm = a2m_edge_attr
    zeros = jnp.zeros((RPAD, HD), jnp.float32)

    segA2M, p = _sc_p5(a_xn, attrAm, srcAm, dstAm, wAm,
                       srcA, dstA, zeros)
    out_e = _tc_seout(p, W_e, a2a_edge_attr, 1000)
    out_mx = _tc_mout(m_xn, Wq, Wk, Wv, Wo, segA2M, W_a2m, m_x, 512)
    segAV2, segM2A = _sc_p4(av2t, en, srcA, dstA, wA,
                            m_xn, attrM, srcM, dstM, wM, zeros)
    segU, segB0 = _sc_p1(a_xn, en, srcA, dstA, wA, v0, zeros)
    out_ax = _tc_axout(segU, segM2A, W_s1, W_m2a, a_x, 1000)
    segB1, segB2 = _sc_p2(a_xn, en, srcA, dstA, wA, v1, v2, zeros)
    segAV0, segAV1 = _sc_p3(av0t, av1t, en, srcA, dstA, wA, zeros)
    out_avec = _tc_avecout(segAV0, segAV1, segAV2, segB0, segB1, segB2,
                           W_s2, a_vec, 1000)
    return out_ax, out_mx, out_avec, out_e
